# baseline (speedup 1.0000x reference)
import functools

import jax
import jax.numpy as jnp
from jax import lax
from jax.experimental import pallas as pl
from jax.experimental.pallas import tpu as pltpu
from jax.experimental.pallas import tpu_sc as plsc

NC = 2
NS = 16
NW = NC * NS
IDX_CHUNK = 128
NPLANE = 8


def _make_sc_gather(F, V, D, B):
    assert B % (8 * NW) == 0
    bpw = B // NW
    nchunk = bpw // IDX_CHUNK
    assert bpw % IDX_CHUNK == 0
    mesh = plsc.VectorSubcoreMesh(core_axis_name="c", subcore_axis_name="s")

    @functools.partial(
        pl.kernel,
        mesh=mesh,
        out_type=jax.ShapeDtypeStruct((B, NPLANE, 128), jnp.float32),
        scratch_types=[
            pltpu.VMEM((bpw,), jnp.int32),
            pltpu.VMEM((bpw, D), jnp.float32),
            pltpu.SemaphoreType.DMA,
        ],
        compiler_params=pltpu.CompilerParams(use_tc_tiling_on_sc=False),
    )
    def sc_gather(table_hbm, idx_hbm, out_hbm, idx_v, rows_v, sem):
        wid = lax.axis_index("s") * NC + lax.axis_index("c")
        base = wid * bpw

        def field_body(f, carry):
            pltpu.sync_copy(idx_hbm.at[pl.ds(f * B + base, bpw)], idx_v)
            cps = [
                pltpu.async_copy(
                    table_hbm.at[idx_v.at[pl.ds(r * IDX_CHUNK, IDX_CHUNK)]],
                    rows_v.at[pl.ds(r * IDX_CHUNK, IDX_CHUNK)],
                    sem,
                )
                for r in range(nchunk)
            ]
            for cp in cps:
                cp.wait()
            pltpu.sync_copy(
                rows_v,
                out_hbm.at[pl.ds(base, bpw), f // 4, pl.ds((f % 4) * D, D)])
            return carry

        lax.fori_loop(0, F, field_body, 0)

    return sc_gather


def _tc_dense(block_b, dense_vals, gat, W_dense, W1d, W1gp, W2, wc_d, wc_fm,
              wc_deep, bc2, n_full, tail_lanes):
    B = gat.shape[0]
    cdims = (((1,), (1,)), ((), ()))

    def body(dv_ref, g_ref, wd_ref, w1d_ref, w1g_ref, w2_ref, wcd_ref,
             wcf_ref, wcdeep_ref, bc_ref, out_ref):
        dv = dv_ref[...]
        dense_feat = lax.dot_general(dv, wd_ref[...], cdims)
        lane = lax.broadcasted_iota(jnp.int32, (block_b, 128), 1)
        pieces = [g_ref[:, j, :] for j in range(n_full)]
        pieces.append(
            jnp.where(lane < tail_lanes, g_ref[:, n_full, :], 0.0))
        s = jnp.sum(dense_feat, axis=1, keepdims=True)
        sq = jnp.sum(dense_feat * dense_feat, axis=1, keepdims=True)
        for x in pieces:
            s = s + jnp.sum(x, axis=1, keepdims=True)
            sq = sq + jnp.sum(x * x, axis=1, keepdims=True)
        fm = 0.5 * (s * s - sq)
        h = lax.dot_general(dense_feat, w1d_ref[...], cdims)
        for j, x in enumerate(pieces):
            h = h + lax.dot_general(x, w1g_ref[:, pl.ds(j * 128, 128)], cdims)
        h = jnp.maximum(h, 0.0)
        deep = jnp.maximum(lax.dot_general(h, w2_ref[...], cdims), 0.0)
        logit = (lax.dot_general(dense_feat, wcd_ref[...], cdims)
                 + fm * wcf_ref[0, 0]
                 + lax.dot_general(deep, wcdeep_ref[...], cdims)
                 + bc_ref[0, 0])
        out_ref[...] = logit

    grid = (B // block_b,)
    const2 = lambda i: (0, 0)
    return pl.pallas_call(
        body,
        grid=grid,
        in_specs=[
            pl.BlockSpec((block_b, dense_vals.shape[1]), lambda i: (i, 0)),
            pl.BlockSpec((block_b, NPLANE, 128), lambda i: (i, 0, 0)),
            pl.BlockSpec(W_dense.shape, const2),
            pl.BlockSpec(W1d.shape, const2),
            pl.BlockSpec(W1gp.shape, const2),
            pl.BlockSpec(W2.shape, const2),
            pl.BlockSpec(wc_d.shape, const2),
            pl.BlockSpec(wc_fm.shape, const2),
            pl.BlockSpec(wc_deep.shape, const2),
            pl.BlockSpec(bc2.shape, const2),
        ],
        out_specs=pl.BlockSpec((block_b, 1), lambda i: (i, 0)),
        out_shape=jax.ShapeDtypeStruct((B, 1), jnp.float32),
    )(dense_vals, gat, W_dense, W1d, W1gp, W2, wc_d, wc_fm, wc_deep, bc2)


def kernel(dense_vals, W_emb, W_dense, W1, W2, Wc, bc, sparse_idx):
    F, V, D = W_emb.shape
    B = dense_vals.shape[0]
    H1 = W1.shape[0]
    table = W_emb.reshape(F * V, D)
    flat_idx = (sparse_idx + (jnp.arange(F, dtype=jnp.int32) * V)[:, None]
                ).reshape(F * B)

    gat = _make_sc_gather(F, V, D, B)(table, flat_idx)

    fd = F * D
    n_full = fd // 128
    tail_lanes = fd - n_full * 128
    W1d = W1[:, :D]
    W1gp = jnp.pad(W1[:, D:], ((0, 0), (0, (n_full + 1) * 128 - fd)))
    wc_d = Wc[:, :D]
    wc_fm = Wc[:, D:D + 1]
    wc_deep = Wc[:, D + 1:]
    bc2 = bc.reshape(1, 1)
    return _tc_dense(512, dense_vals, gat, W_dense, W1d, W1gp, W2,
                     wc_d, wc_fm, wc_deep, bc2, n_full, tail_lanes)
